# baseline (speedup 1.0000x reference)
import functools

import jax
import jax.numpy as jnp
from jax import lax
from jax.experimental import pallas as pl
from jax.experimental.pallas import tpu as pltpu
from jax.experimental.pallas import tpu_sc as plsc

N_ITEMS = 16384
EMB_DIM = 64
KNN_K = 10
K_PAD = 16
BLOCK_R = 256
NORM_BLOCK = 1024


def _normalize_body(x_ref, xn_ref):
    x = x_ref[...]
    nrm = jnp.sqrt(jnp.sum(x * x, axis=1, keepdims=True))
    xn_ref[...] = x / nrm


SEGS = 128
SEG_T = 4
N_HALVES = 1


def _extract_topk(sim, idx_src, big, mask_val=-3.0):
    vals, idxs = [], []
    for t in range(KNN_K):
        m = jnp.max(sim, axis=1, keepdims=True)
        eq = sim == m
        idx = jnp.min(jnp.where(eq, idx_src, big), axis=1, keepdims=True)
        vals.append(m)
        idxs.append(idx)
        if t < KNN_K - 1:
            sim = jnp.where(idx_src == idx, mask_val, sim)
    return jnp.concatenate(vals, axis=1), jnp.concatenate(idxs, axis=1)


def _fast_rsqrt(x):
    y = jnp.full(x.shape, 0.3, dtype=jnp.float32)
    for _ in range(4):
        y = y * (1.5 - 0.5 * x * y * y)
    return y


def _sc_geometry():
    info = plsc.get_sparse_core_info()
    nc, ns, L = info.num_cores, info.num_subcores, info.num_lanes
    return nc, ns, L, nc * ns


def _rinv_sc():
    nc, ns, L, nw = _sc_geometry()
    rows_per_w = N_ITEMS // nw
    mesh = plsc.VectorSubcoreMesh(core_axis_name="c", subcore_axis_name="s")

    @functools.partial(
        pl.kernel, mesh=mesh,
        out_type=jax.ShapeDtypeStruct((N_ITEMS,), jnp.float32),
        scratch_types=[pltpu.VMEM((rows_per_w,), jnp.float32)],
    )
    def rinv_kernel(rinv_hbm, buf_v):
        wid = lax.axis_index("s") * nc + lax.axis_index("c")
        row0 = wid * rows_per_w
        deg = jnp.zeros((L,), jnp.float32)
        for _ in range(KNN_K):
            deg = deg + 1.0
        rinv = _fast_rsqrt(deg + 1e-07)

        @plsc.parallel_loop(0, rows_per_w // L, step=1)
        def _fill(i):
            buf_v[pl.ds(i * L, L)] = rinv

        pltpu.sync_copy(buf_v, rinv_hbm.at[pl.ds(row0, rows_per_w)])

    return rinv_kernel()


def _edges_sc(cols_flat, rinv):
    nc, ns, L, nw = _sc_geometry()
    rows_per_w = N_ITEMS // nw
    edges_per_w = rows_per_w * KNN_K
    n_edges = N_ITEMS * KNN_K
    mesh = plsc.VectorSubcoreMesh(core_axis_name="c", subcore_axis_name="s")

    @functools.partial(
        pl.kernel, mesh=mesh,
        out_type=[
            jax.ShapeDtypeStruct((n_edges,), jnp.int32),
            jax.ShapeDtypeStruct((n_edges,), jnp.float32),
        ],
        scratch_types=[
            pltpu.VMEM((edges_per_w,), jnp.int32),
            pltpu.VMEM((edges_per_w,), jnp.int32),
            pltpu.VMEM((edges_per_w,), jnp.float32),
            pltpu.VMEM((edges_per_w,), jnp.float32),
            pltpu.SemaphoreType.DMA,
        ],
    )
    def edges_kernel(cols_hbm, rinv_hbm, rows_out, vals_out,
                     r_v, c_v, vr_v, vc_v, sem):
        wid = lax.axis_index("s") * nc + lax.axis_index("c")
        row0 = wid * rows_per_w
        e0 = wid * edges_per_w
        pltpu.sync_copy(cols_hbm.at[pl.ds(e0, edges_per_w)], c_v)
        lane = lax.iota(jnp.int32, L)

        @plsc.parallel_loop(0, edges_per_w // L, step=1)
        def _rows(j):
            e = j * L + lane
            r_loc = lax.shift_right_logical(e * 52429, 19)
            r_v[pl.ds(j * L, L)] = r_loc + row0

        pltpu.async_copy(rinv_hbm.at[r_v], vr_v, sem).wait()
        pltpu.async_copy(rinv_hbm.at[c_v], vc_v, sem).wait()

        @plsc.parallel_loop(0, edges_per_w // L, step=1)
        def _mul(j):
            sl = pl.ds(j * L, L)
            vr_v[sl] = vr_v[sl] * vc_v[sl]

        pltpu.sync_copy(r_v, rows_out.at[pl.ds(e0, edges_per_w)])
        pltpu.sync_copy(vr_v, vals_out.at[pl.ds(e0, edges_per_w)])

    return edges_kernel(cols_flat, rinv)


def _topk_body(xb_ref, xt_ref, val_ref, idx_ref):
    xb = xb_ref[...]
    xt = xt_ref[...]

    half = N_ITEMS // N_HALVES
    seg_w = half // SEGS
    wpos = jax.lax.broadcasted_iota(jnp.int32, (BLOCK_R, seg_w, SEGS), 1)
    soff = jax.lax.broadcasted_iota(jnp.int32, (BLOCK_R, SEGS), 1)
    cvals, cidxs, tails = [], [], []
    for h in range(N_HALVES):
        simh = jnp.dot(xb, xt[:, h * half:(h + 1) * half],
                       preferred_element_type=jnp.float32)
        sim3 = simh.reshape(BLOCK_R, seg_w, SEGS)
        base = h * half + soff
        for t in range(SEG_T):
            m = jnp.max(sim3, axis=1, keepdims=True)
            eq = sim3 == m
            iw = jnp.min(jnp.where(eq, wpos, seg_w), axis=1, keepdims=True)
            cvals.append(m[:, 0, :])
            cidxs.append(iw[:, 0, :] * SEGS + base)
            if t < SEG_T - 1:
                sim3 = jnp.where(wpos == iw, -3.0, sim3)
        tails.append(cvals[-1])
    cand_v = jnp.concatenate(cvals, axis=1)
    cand_i = jnp.concatenate(cidxs, axis=1)

    vals, idxs = _extract_topk(cand_v, cand_i, N_ITEMS)

    pad_v = jnp.zeros((BLOCK_R, K_PAD - KNN_K), dtype=jnp.float32)
    pad_i = jnp.zeros((BLOCK_R, K_PAD - KNN_K), dtype=jnp.int32)
    val_ref[...] = jnp.concatenate([vals, pad_v], axis=1)
    idx_ref[...] = jnp.concatenate([idxs, pad_i], axis=1)

    v10 = vals[:, KNN_K - 1:KNN_K]
    flag = jnp.any(jnp.concatenate(tails, axis=1) >= v10)

    @pl.when(flag)
    def _fallback():
        simf = jnp.dot(xb, xt, preferred_element_type=jnp.float32)
        col = jax.lax.broadcasted_iota(jnp.int32, (BLOCK_R, N_ITEMS), 1)
        fv, fi = _extract_topk(simf, col, N_ITEMS)
        val_ref[...] = jnp.concatenate([fv, pad_v], axis=1)
        idx_ref[...] = jnp.concatenate([fi, pad_i], axis=1)


def kernel(mm_embeddings):
    n = N_ITEMS
    xn = pl.pallas_call(
        _normalize_body,
        grid=(n // NORM_BLOCK,),
        in_specs=[pl.BlockSpec((NORM_BLOCK, EMB_DIM), lambda i: (i, 0))],
        out_specs=pl.BlockSpec((NORM_BLOCK, EMB_DIM), lambda i: (i, 0)),
        out_shape=jax.ShapeDtypeStruct((n, EMB_DIM), jnp.float32),
    )(mm_embeddings)
    xt = xn.T

    vals, idxs = pl.pallas_call(
        _topk_body,
        grid=(n // BLOCK_R,),
        in_specs=[
            pl.BlockSpec((BLOCK_R, EMB_DIM), lambda i: (i, 0)),
            pl.BlockSpec((EMB_DIM, n), lambda i: (0, 0)),
        ],
        out_specs=[
            pl.BlockSpec((BLOCK_R, K_PAD), lambda i: (i, 0)),
            pl.BlockSpec((BLOCK_R, K_PAD), lambda i: (i, 0)),
        ],
        out_shape=[
            jax.ShapeDtypeStruct((n, K_PAD), jnp.float32),
            jax.ShapeDtypeStruct((n, K_PAD), jnp.int32),
        ],
        compiler_params=pltpu.CompilerParams(
            dimension_semantics=("parallel",),
        ),
    )(xn, xt)

    knn_val = vals[:, :KNN_K]
    cols = idxs[:, :KNN_K].reshape(-1)
    rinv = _rinv_sc()
    rows_o, values = _edges_sc(cols, rinv)
    indices = jnp.stack((rows_o, cols), axis=0)
    return knn_val, indices, values


# speedup vs baseline: 1.3772x; 1.3772x over previous
import functools

import jax
import jax.numpy as jnp
from jax import lax
from jax.experimental import pallas as pl
from jax.experimental.pallas import tpu as pltpu
from jax.experimental.pallas import tpu_sc as plsc

N_ITEMS = 16384
EMB_DIM = 64
KNN_K = 10
K_PAD = 16
BLOCK_R = 64
NORM_BLOCK = 1024


def _normalize_body(x_ref, xn_ref):
    x = x_ref[...]
    nrm = jnp.sqrt(jnp.sum(x * x, axis=1, keepdims=True))
    xn_ref[...] = x / nrm


SEGS = 128
SEG_T = 4
N_HALVES = 1


def _extract_topk(sim, idx_src, big, mask_val=-3.0):
    vals, idxs = [], []
    for t in range(KNN_K):
        m = jnp.max(sim, axis=1, keepdims=True)
        eq = sim == m
        idx = jnp.min(jnp.where(eq, idx_src, big), axis=1, keepdims=True)
        vals.append(m)
        idxs.append(idx)
        if t < KNN_K - 1:
            sim = jnp.where(idx_src == idx, mask_val, sim)
    return jnp.concatenate(vals, axis=1), jnp.concatenate(idxs, axis=1)


def _fast_rsqrt(x):
    y = jnp.full(x.shape, 0.3, dtype=jnp.float32)
    for _ in range(4):
        y = y * (1.5 - 0.5 * x * y * y)
    return y


def _sc_geometry():
    info = plsc.get_sparse_core_info()
    nc, ns, L = info.num_cores, info.num_subcores, info.num_lanes
    return nc, ns, L, nc * ns


def _rinv_sc():
    nc, ns, L, nw = _sc_geometry()
    rows_per_w = N_ITEMS // nw
    mesh = plsc.VectorSubcoreMesh(core_axis_name="c", subcore_axis_name="s")

    @functools.partial(
        pl.kernel, mesh=mesh,
        out_type=jax.ShapeDtypeStruct((N_ITEMS,), jnp.float32),
        scratch_types=[pltpu.VMEM((rows_per_w,), jnp.float32)],
    )
    def rinv_kernel(rinv_hbm, buf_v):
        wid = lax.axis_index("s") * nc + lax.axis_index("c")
        row0 = wid * rows_per_w
        deg = jnp.zeros((L,), jnp.float32)
        for _ in range(KNN_K):
            deg = deg + 1.0
        rinv = _fast_rsqrt(deg + 1e-07)

        @plsc.parallel_loop(0, rows_per_w // L, step=1)
        def _fill(i):
            buf_v[pl.ds(i * L, L)] = rinv

        pltpu.sync_copy(buf_v, rinv_hbm.at[pl.ds(row0, rows_per_w)])

    return rinv_kernel()


def _edges_sc(cols_flat, rinv):
    nc, ns, L, nw = _sc_geometry()
    rows_per_w = N_ITEMS // nw
    edges_per_w = rows_per_w * KNN_K
    n_edges = N_ITEMS * KNN_K
    mesh = plsc.VectorSubcoreMesh(core_axis_name="c", subcore_axis_name="s")

    @functools.partial(
        pl.kernel, mesh=mesh,
        out_type=[
            jax.ShapeDtypeStruct((n_edges,), jnp.int32),
            jax.ShapeDtypeStruct((n_edges,), jnp.float32),
        ],
        scratch_types=[
            pltpu.VMEM((edges_per_w,), jnp.int32),
            pltpu.VMEM((edges_per_w,), jnp.int32),
            pltpu.VMEM((edges_per_w,), jnp.float32),
            pltpu.VMEM((edges_per_w,), jnp.float32),
            pltpu.SemaphoreType.DMA,
        ],
    )
    def edges_kernel(cols_hbm, rinv_hbm, rows_out, vals_out,
                     r_v, c_v, vr_v, vc_v, sem):
        wid = lax.axis_index("s") * nc + lax.axis_index("c")
        row0 = wid * rows_per_w
        e0 = wid * edges_per_w
        pltpu.sync_copy(cols_hbm.at[pl.ds(e0, edges_per_w)], c_v)
        lane = lax.iota(jnp.int32, L)

        @plsc.parallel_loop(0, edges_per_w // L, step=1)
        def _rows(j):
            e = j * L + lane
            r_loc = lax.shift_right_logical(e * 52429, 19)
            r_v[pl.ds(j * L, L)] = r_loc + row0

        pltpu.async_copy(rinv_hbm.at[r_v], vr_v, sem).wait()
        pltpu.async_copy(rinv_hbm.at[c_v], vc_v, sem).wait()

        @plsc.parallel_loop(0, edges_per_w // L, step=1)
        def _mul(j):
            sl = pl.ds(j * L, L)
            vr_v[sl] = vr_v[sl] * vc_v[sl]

        pltpu.sync_copy(r_v, rows_out.at[pl.ds(e0, edges_per_w)])
        pltpu.sync_copy(vr_v, vals_out.at[pl.ds(e0, edges_per_w)])

    return edges_kernel(cols_flat, rinv)


def _topk_body(xb_ref, xt_ref, val_ref, idx_ref):
    xb = xb_ref[...]
    xt = xt_ref[...]

    half = N_ITEMS // N_HALVES
    seg_w = half // SEGS
    wpos = jax.lax.broadcasted_iota(jnp.int32, (BLOCK_R, seg_w, SEGS), 1)
    soff = jax.lax.broadcasted_iota(jnp.int32, (BLOCK_R, SEGS), 1)
    cvals, cidxs, tails = [], [], []
    for h in range(N_HALVES):
        simh = jnp.dot(xb, xt[:, h * half:(h + 1) * half],
                       preferred_element_type=jnp.float32)
        sim3 = simh.reshape(BLOCK_R, seg_w, SEGS)
        base = h * half + soff
        for t in range(SEG_T):
            m = jnp.max(sim3, axis=1, keepdims=True)
            eq = sim3 == m
            iw = jnp.min(jnp.where(eq, wpos, seg_w), axis=1, keepdims=True)
            cvals.append(m[:, 0, :])
            cidxs.append(iw[:, 0, :] * SEGS + base)
            if t < SEG_T - 1:
                sim3 = jnp.where(wpos == iw, -3.0, sim3)
        tails.append(cvals[-1])
    cand_v = jnp.concatenate(cvals, axis=1)
    cand_i = jnp.concatenate(cidxs, axis=1)

    vals, idxs = _extract_topk(cand_v, cand_i, N_ITEMS)

    pad_v = jnp.zeros((BLOCK_R, K_PAD - KNN_K), dtype=jnp.float32)
    pad_i = jnp.zeros((BLOCK_R, K_PAD - KNN_K), dtype=jnp.int32)
    val_ref[...] = jnp.concatenate([vals, pad_v], axis=1)
    idx_ref[...] = jnp.concatenate([idxs, pad_i], axis=1)

    v10 = vals[:, KNN_K - 1:KNN_K]
    flag = jnp.any(jnp.concatenate(tails, axis=1) >= v10)

    @pl.when(flag)
    def _fallback():
        simf = jnp.dot(xb, xt, preferred_element_type=jnp.float32)
        col = jax.lax.broadcasted_iota(jnp.int32, (BLOCK_R, N_ITEMS), 1)
        fv, fi = _extract_topk(simf, col, N_ITEMS)
        val_ref[...] = jnp.concatenate([fv, pad_v], axis=1)
        idx_ref[...] = jnp.concatenate([fi, pad_i], axis=1)


def kernel(mm_embeddings):
    n = N_ITEMS
    xn = pl.pallas_call(
        _normalize_body,
        grid=(n // NORM_BLOCK,),
        in_specs=[pl.BlockSpec((NORM_BLOCK, EMB_DIM), lambda i: (i, 0))],
        out_specs=pl.BlockSpec((NORM_BLOCK, EMB_DIM), lambda i: (i, 0)),
        out_shape=jax.ShapeDtypeStruct((n, EMB_DIM), jnp.float32),
    )(mm_embeddings)
    xt = xn.T

    vals, idxs = pl.pallas_call(
        _topk_body,
        grid=(n // BLOCK_R,),
        in_specs=[
            pl.BlockSpec((BLOCK_R, EMB_DIM), lambda i: (i, 0)),
            pl.BlockSpec((EMB_DIM, n), lambda i: (0, 0)),
        ],
        out_specs=[
            pl.BlockSpec((BLOCK_R, K_PAD), lambda i: (i, 0)),
            pl.BlockSpec((BLOCK_R, K_PAD), lambda i: (i, 0)),
        ],
        out_shape=[
            jax.ShapeDtypeStruct((n, K_PAD), jnp.float32),
            jax.ShapeDtypeStruct((n, K_PAD), jnp.int32),
        ],
        compiler_params=pltpu.CompilerParams(
            dimension_semantics=("parallel",),
        ),
    )(xn, xt)

    knn_val = vals[:, :KNN_K]
    cols = idxs[:, :KNN_K].reshape(-1)
    rinv = _rinv_sc()
    rows_o, values = _edges_sc(cols, rinv)
    indices = jnp.stack((rows_o, cols), axis=0)
    return knn_val, indices, values


# speedup vs baseline: 1.6847x; 1.2233x over previous
import functools

import jax
import jax.numpy as jnp
from jax import lax
from jax.experimental import pallas as pl
from jax.experimental.pallas import tpu as pltpu
from jax.experimental.pallas import tpu_sc as plsc

N_ITEMS = 16384
EMB_DIM = 64
KNN_K = 10
K_PAD = 16
BLOCK_R = 128
NORM_BLOCK = 1024


def _normalize_body(x_ref, xn_ref):
    x = x_ref[...]
    nrm = jnp.sqrt(jnp.sum(x * x, axis=1, keepdims=True))
    xn_ref[...] = x / nrm


SEGS = 128
SEG_T = 4
N_HALVES = 1


def _extract_topk(sim, idx_src, big, mask_val=-3.0):
    vals, idxs = [], []
    for t in range(KNN_K):
        m = jnp.max(sim, axis=1, keepdims=True)
        eq = sim == m
        idx = jnp.min(jnp.where(eq, idx_src, big), axis=1, keepdims=True)
        vals.append(m)
        idxs.append(idx)
        if t < KNN_K - 1:
            sim = jnp.where(idx_src == idx, mask_val, sim)
    return jnp.concatenate(vals, axis=1), jnp.concatenate(idxs, axis=1)


def _fast_rsqrt(x):
    y = jnp.full(x.shape, 0.3, dtype=jnp.float32)
    for _ in range(4):
        y = y * (1.5 - 0.5 * x * y * y)
    return y


def _sc_geometry():
    info = plsc.get_sparse_core_info()
    nc, ns, L = info.num_cores, info.num_subcores, info.num_lanes
    return nc, ns, L, nc * ns


def _rinv_sc():
    nc, ns, L, nw = _sc_geometry()
    rows_per_w = N_ITEMS // nw
    mesh = plsc.VectorSubcoreMesh(core_axis_name="c", subcore_axis_name="s")

    @functools.partial(
        pl.kernel, mesh=mesh,
        out_type=jax.ShapeDtypeStruct((N_ITEMS,), jnp.float32),
        scratch_types=[pltpu.VMEM((rows_per_w,), jnp.float32)],
    )
    def rinv_kernel(rinv_hbm, buf_v):
        wid = lax.axis_index("s") * nc + lax.axis_index("c")
        row0 = wid * rows_per_w
        deg = jnp.zeros((L,), jnp.float32)
        for _ in range(KNN_K):
            deg = deg + 1.0
        rinv = _fast_rsqrt(deg + 1e-07)

        @plsc.parallel_loop(0, rows_per_w // L, step=1)
        def _fill(i):
            buf_v[pl.ds(i * L, L)] = rinv

        pltpu.sync_copy(buf_v, rinv_hbm.at[pl.ds(row0, rows_per_w)])

    return rinv_kernel()


def _edges_sc(cols_flat, rinv):
    nc, ns, L, nw = _sc_geometry()
    rows_per_w = N_ITEMS // nw
    edges_per_w = rows_per_w * KNN_K
    n_edges = N_ITEMS * KNN_K
    mesh = plsc.VectorSubcoreMesh(core_axis_name="c", subcore_axis_name="s")

    @functools.partial(
        pl.kernel, mesh=mesh,
        out_type=[
            jax.ShapeDtypeStruct((n_edges,), jnp.int32),
            jax.ShapeDtypeStruct((n_edges,), jnp.float32),
        ],
        scratch_types=[
            pltpu.VMEM((edges_per_w,), jnp.int32),
            pltpu.VMEM((edges_per_w,), jnp.int32),
            pltpu.VMEM((edges_per_w,), jnp.float32),
            pltpu.VMEM((edges_per_w,), jnp.float32),
            pltpu.SemaphoreType.DMA,
        ],
    )
    def edges_kernel(cols_hbm, rinv_hbm, rows_out, vals_out,
                     r_v, c_v, vr_v, vc_v, sem):
        wid = lax.axis_index("s") * nc + lax.axis_index("c")
        row0 = wid * rows_per_w
        e0 = wid * edges_per_w
        pltpu.sync_copy(cols_hbm.at[pl.ds(e0, edges_per_w)], c_v)
        lane = lax.iota(jnp.int32, L)

        @plsc.parallel_loop(0, edges_per_w // L, step=1)
        def _rows(j):
            e = j * L + lane
            r_loc = lax.shift_right_logical(e * 52429, 19)
            r_v[pl.ds(j * L, L)] = r_loc + row0

        pltpu.async_copy(rinv_hbm.at[r_v], vr_v, sem).wait()
        pltpu.async_copy(rinv_hbm.at[c_v], vc_v, sem).wait()

        @plsc.parallel_loop(0, edges_per_w // L, step=1)
        def _mul(j):
            sl = pl.ds(j * L, L)
            vr_v[sl] = vr_v[sl] * vc_v[sl]

        pltpu.sync_copy(r_v, rows_out.at[pl.ds(e0, edges_per_w)])
        pltpu.sync_copy(vr_v, vals_out.at[pl.ds(e0, edges_per_w)])

    return edges_kernel(cols_flat, rinv)


def _topk_body(xb_ref, xt_ref, val_ref, idx_ref):
    xb = xb_ref[...]
    xt = xt_ref[...]

    half = N_ITEMS // N_HALVES
    seg_w = half // SEGS
    wpos = jax.lax.broadcasted_iota(jnp.int32, (BLOCK_R, seg_w, SEGS), 1)
    soff = jax.lax.broadcasted_iota(jnp.int32, (BLOCK_R, SEGS), 1)
    cvals, cidxs, tails = [], [], []
    for h in range(N_HALVES):
        simh = jnp.dot(xb, xt[:, h * half:(h + 1) * half],
                       preferred_element_type=jnp.float32)
        sim3 = simh.reshape(BLOCK_R, seg_w, SEGS)
        base = h * half + soff
        for t in range(SEG_T):
            v, iw, w = sim3, wpos, seg_w
            while w > 1:
                hw = w // 2
                ta = v[:, :hw, :] >= v[:, hw:, :]
                v = jnp.where(ta, v[:, :hw, :], v[:, hw:, :])
                iw = jnp.where(ta, iw[:, :hw, :], iw[:, hw:, :])
                w = hw
            cvals.append(v[:, 0, :])
            cidxs.append(iw[:, 0, :] * SEGS + base)
            if t < SEG_T - 1:
                sim3 = jnp.where(wpos == iw, -3.0, sim3)
        tails.append(cvals[-1])
    cand_v = jnp.concatenate(cvals, axis=1)
    cand_i = jnp.concatenate(cidxs, axis=1)

    vals, idxs = _extract_topk(cand_v, cand_i, N_ITEMS)

    pad_v = jnp.zeros((BLOCK_R, K_PAD - KNN_K), dtype=jnp.float32)
    pad_i = jnp.zeros((BLOCK_R, K_PAD - KNN_K), dtype=jnp.int32)
    val_ref[...] = jnp.concatenate([vals, pad_v], axis=1)
    idx_ref[...] = jnp.concatenate([idxs, pad_i], axis=1)

    v10 = vals[:, KNN_K - 1:KNN_K]
    flag = jnp.any(jnp.concatenate(tails, axis=1) >= v10)

    @pl.when(flag)
    def _fallback():
        simf = jnp.dot(xb, xt, preferred_element_type=jnp.float32)
        col = jax.lax.broadcasted_iota(jnp.int32, (BLOCK_R, N_ITEMS), 1)
        fv, fi = _extract_topk(simf, col, N_ITEMS)
        val_ref[...] = jnp.concatenate([fv, pad_v], axis=1)
        idx_ref[...] = jnp.concatenate([fi, pad_i], axis=1)


def kernel(mm_embeddings):
    n = N_ITEMS
    xn = pl.pallas_call(
        _normalize_body,
        grid=(n // NORM_BLOCK,),
        in_specs=[pl.BlockSpec((NORM_BLOCK, EMB_DIM), lambda i: (i, 0))],
        out_specs=pl.BlockSpec((NORM_BLOCK, EMB_DIM), lambda i: (i, 0)),
        out_shape=jax.ShapeDtypeStruct((n, EMB_DIM), jnp.float32),
    )(mm_embeddings)
    xt = xn.T

    vals, idxs = pl.pallas_call(
        _topk_body,
        grid=(n // BLOCK_R,),
        in_specs=[
            pl.BlockSpec((BLOCK_R, EMB_DIM), lambda i: (i, 0)),
            pl.BlockSpec((EMB_DIM, n), lambda i: (0, 0)),
        ],
        out_specs=[
            pl.BlockSpec((BLOCK_R, K_PAD), lambda i: (i, 0)),
            pl.BlockSpec((BLOCK_R, K_PAD), lambda i: (i, 0)),
        ],
        out_shape=[
            jax.ShapeDtypeStruct((n, K_PAD), jnp.float32),
            jax.ShapeDtypeStruct((n, K_PAD), jnp.int32),
        ],
        compiler_params=pltpu.CompilerParams(
            dimension_semantics=("parallel",),
        ),
    )(xn, xt)

    knn_val = vals[:, :KNN_K]
    cols = idxs[:, :KNN_K].reshape(-1)
    rinv = _rinv_sc()
    rows_o, values = _edges_sc(cols, rinv)
    indices = jnp.stack((rows_o, cols), axis=0)
    return knn_val, indices, values
